# Initial kernel scaffold; baseline (speedup 1.0000x reference)
import jax
import jax.numpy as jnp
from jax.experimental import pallas as pl


def kernel(x, edge_index, W0, b0, jk1_Wih_f, jk1_Whh_f, jk1_bih_f, jk1_bhh_f, jk1_Wih_b, jk1_Whh_b, jk1_bih_b, jk1_bhh_b, jk1_att_W, jk1_att_b, W1, b1, jk2_Wih_f, jk2_Whh_f, jk2_bih_f, jk2_bhh_f, jk2_Wih_b, jk2_Whh_b, jk2_bih_b, jk2_bhh_b, jk2_att_W, jk2_att_b, W2, b2):
    raise NotImplementedError("write your pallas kernel here")


# speedup vs baseline: 10.9908x; 10.9908x over previous
import functools

import jax
import jax.numpy as jnp
from jax import lax
from jax.experimental import pallas as pl
from jax.experimental.pallas import tpu as pltpu
from jax.experimental.pallas import tpu_sc as plsc

N = 10000
E = 320000
F = 128
NCLS = 40
NC, NS = 2, 16
NW = NC * NS
EPW = E // NW
CH = 80
NP = 10240
RPT = NP // NS
RB = 1000

@functools.lru_cache(maxsize=None)
def _mesh():
    return plsc.VectorSubcoreMesh(core_axis_name="c", subcore_axis_name="s")


@functools.lru_cache(maxsize=None)
def _make_agg(D):
    @functools.partial(
        pl.kernel,
        out_type=jax.ShapeDtypeStruct((NC, NP, D), jnp.float32),
        mesh=_mesh(),
        scratch_types=[
            pltpu.VMEM((CH,), jnp.int32),
            pltpu.VMEM((CH,), jnp.int32),
            pltpu.VMEM((CH, D), jnp.float32),
            pltpu.VMEM_SHARED((NP, D), jnp.float32),
            pltpu.SemaphoreType.DMA,
        ],
    )
    def agg(src_hbm, dst_hbm, y_hbm, zero_hbm, part_hbm, idx_s, idx_d, rows, acc, sem):
        cid = lax.axis_index("c")
        sid = lax.axis_index("s")
        wid = sid * NC + cid
        r0 = sid * RPT
        pltpu.sync_copy(zero_hbm.at[pl.ds(r0, RPT)], acc.at[pl.ds(r0, RPT)])
        plsc.subcore_barrier()

        def body(i, carry):
            base = wid * EPW + i * CH
            pltpu.sync_copy(src_hbm.at[pl.ds(base, CH)], idx_s)
            pltpu.async_copy(y_hbm.at[idx_s], rows, sem).wait()
            pltpu.sync_copy(dst_hbm.at[pl.ds(base, CH)], idx_d)
            pltpu.sync_copy(rows, acc.at[idx_d], add=True)
            return carry

        lax.fori_loop(0, EPW // CH, body, 0)
        plsc.subcore_barrier()
        pltpu.sync_copy(acc.at[pl.ds(r0, RPT)], part_hbm.at[cid, pl.ds(r0, RPT)])

    return agg


def _agg128(src, dst, y, zero):
    return _make_agg(F)(src, dst, y, zero)


@functools.lru_cache(maxsize=None)
def _make_deg():
    @functools.partial(
        pl.kernel,
        out_type=jax.ShapeDtypeStruct((NC, NP), jnp.float32),
        mesh=_mesh(),
        scratch_types=[
            pltpu.VMEM((CH,), jnp.int32),
            pltpu.VMEM((CH,), jnp.float32),
            pltpu.VMEM_SHARED((NP,), jnp.float32),
        ],
    )
    def deg(dst_hbm, ones_hbm, zero_hbm, part_hbm, idx_d, ones_v, acc):
        cid = lax.axis_index("c")
        sid = lax.axis_index("s")
        wid = sid * NC + cid
        r0 = sid * RPT
        pltpu.sync_copy(zero_hbm.at[pl.ds(r0, RPT)], acc.at[pl.ds(r0, RPT)])
        pltpu.sync_copy(ones_hbm, ones_v)
        plsc.subcore_barrier()

        def body(i, carry):
            base = wid * EPW + i * CH
            pltpu.sync_copy(dst_hbm.at[pl.ds(base, CH)], idx_d)
            pltpu.sync_copy(ones_v, acc.at[idx_d], add=True)
            return carry

        lax.fori_loop(0, EPW // CH, body, 0)
        plsc.subcore_barrier()
        pltpu.sync_copy(acc.at[pl.ds(r0, RPT)], part_hbm.at[cid, pl.ds(r0, RPT)])

    return deg


def _deg_part(dst, ones, zero):
    return _make_deg()(dst, ones, zero)


def _dinv_of(degp_ref):
    deg = degp_ref[0, :, 0:1] + degp_ref[1, :, 0:1] + 1.0
    return lax.rsqrt(deg)


def _k1(x_ref, w0_ref, degp_ref, y0_ref):
    dinv = _dinv_of(degp_ref)
    y0_ref[...] = jnp.dot(x_ref[...], w0_ref[...],
                          preferred_element_type=jnp.float32) * dinv


def _k2(p_ref, y0_ref, degp_ref, b0_ref, w1_ref, h_ref, y1_ref):
    dinv = _dinv_of(degp_ref)
    h = jnp.maximum(dinv * (p_ref[0] + p_ref[1] + y0_ref[...]) + b0_ref[...], 0.0)
    h_ref[...] = h
    y1_ref[...] = jnp.dot(h, w1_ref[...], preferred_element_type=jnp.float32) * dinv


def _lstm_step0(xt, WiT, b):
    g = jnp.dot(xt, WiT, preferred_element_type=jnp.float32) + b
    c = jax.nn.sigmoid(g[:, 0:F]) * jnp.tanh(g[:, 2 * F:3 * F])
    return jax.nn.sigmoid(g[:, 3 * F:4 * F]) * jnp.tanh(c), c


def _lstm_step(xt, hp, cp, WiT, WhT, b):
    g = (jnp.dot(xt, WiT, preferred_element_type=jnp.float32)
         + jnp.dot(hp, WhT, preferred_element_type=jnp.float32) + b)
    c = (jax.nn.sigmoid(g[:, F:2 * F]) * cp
         + jax.nn.sigmoid(g[:, 0:F]) * jnp.tanh(g[:, 2 * F:3 * F]))
    return jax.nn.sigmoid(g[:, 3 * F:4 * F]) * jnp.tanh(c), c


def _k3(q_ref, y1_ref, h_ref, degp_ref, b1_ref, wif_ref, whf_ref, bf_ref,
        wib_ref, whb_ref, bb_ref, awf_ref, awb_ref, z_ref):
    dinv = _dinv_of(degp_ref)
    h = h_ref[...]
    h2 = jnp.maximum(dinv * (q_ref[0] + q_ref[1] + y1_ref[...]) + b1_ref[...], 0.0)
    hf0, cf0 = _lstm_step0(h, wif_ref[...], bf_ref[...])
    hf1, _ = _lstm_step(h2, hf0, cf0, wif_ref[...], whf_ref[...], bf_ref[...])
    hb1, cb1 = _lstm_step0(h2, wib_ref[...], bb_ref[...])
    hb0, _ = _lstm_step(h, hb1, cb1, wib_ref[...], whb_ref[...], bb_ref[...])
    awf = awf_ref[...]
    awb = awb_ref[...]
    a0 = (jnp.sum(hf0 * awf, axis=1, keepdims=True)
          + jnp.sum(hb0 * awb, axis=1, keepdims=True))
    a1 = (jnp.sum(hf1 * awf, axis=1, keepdims=True)
          + jnp.sum(hb1 * awb, axis=1, keepdims=True))
    m = jnp.maximum(a0, a1)
    e0 = jnp.exp(a0 - m)
    e1 = jnp.exp(a1 - m)
    ho = (e0 * h + e1 * h2) / (e0 + e1)
    z_ref[...] = ho * dinv


def _k4(r_ref, z_ref, degp_ref, w2_ref, b2_ref, out_ref):
    dinv = _dinv_of(degp_ref)
    t = dinv * (r_ref[0] + r_ref[1] + z_ref[...])
    l = jnp.dot(t, w2_ref[...], preferred_element_type=jnp.float32) + b2_ref[...]
    m = jnp.max(l, axis=1, keepdims=True)
    lse = jnp.log(jnp.sum(jnp.exp(l - m), axis=1, keepdims=True))
    out_ref[...] = l - m - lse


def _row_spec(d):
    return pl.BlockSpec((RB, d), lambda r: (r, 0))


def _part_spec(d):
    return pl.BlockSpec((NC, RB, d), lambda r: (0, r, 0))


def _full_spec(a, b):
    return pl.BlockSpec((a, b), lambda r: (0, 0))


_GRID = (N // RB,)

_tc1 = pl.pallas_call(
    _k1, grid=_GRID,
    in_specs=[_row_spec(F), _full_spec(F, F), _part_spec(1)],
    out_specs=_row_spec(F),
    out_shape=jax.ShapeDtypeStruct((N, F), jnp.float32))

_tc2 = pl.pallas_call(
    _k2, grid=_GRID,
    in_specs=[_part_spec(F), _row_spec(F), _part_spec(1), _full_spec(1, F),
              _full_spec(F, F)],
    out_specs=[_row_spec(F), _row_spec(F)],
    out_shape=[jax.ShapeDtypeStruct((N, F), jnp.float32),
               jax.ShapeDtypeStruct((N, F), jnp.float32)])

_tc3 = pl.pallas_call(
    _k3, grid=_GRID,
    in_specs=[_part_spec(F), _row_spec(F), _row_spec(F), _part_spec(1),
              _full_spec(1, F),
              _full_spec(F, 4 * F), _full_spec(F, 4 * F), _full_spec(1, 4 * F),
              _full_spec(F, 4 * F), _full_spec(F, 4 * F), _full_spec(1, 4 * F),
              _full_spec(1, F), _full_spec(1, F)],
    out_specs=_row_spec(F),
    out_shape=jax.ShapeDtypeStruct((N, F), jnp.float32))

_tc4 = pl.pallas_call(
    _k4, grid=_GRID,
    in_specs=[_part_spec(F), _row_spec(F), _part_spec(1),
              _full_spec(F, NCLS), _full_spec(1, NCLS)],
    out_specs=pl.BlockSpec((RB, NCLS), lambda r: (r, 0)),
    out_shape=jax.ShapeDtypeStruct((N, NCLS), jnp.float32))


def kernel(x, edge_index, W0, b0, jk1_Wih_f, jk1_Whh_f, jk1_bih_f, jk1_bhh_f,
           jk1_Wih_b, jk1_Whh_b, jk1_bih_b, jk1_bhh_b, jk1_att_W, jk1_att_b,
           W1, b1, jk2_Wih_f, jk2_Whh_f, jk2_bih_f, jk2_bhh_f, jk2_Wih_b,
           jk2_Whh_b, jk2_bih_b, jk2_bhh_b, jk2_att_W, jk2_att_b, W2, b2):
    src = edge_index[0]
    dst = edge_index[1]
    z1 = jnp.zeros((NP,), jnp.float32)
    z128 = jnp.zeros((NP, F), jnp.float32)
    ones = jnp.ones((CH,), jnp.float32)

    degp = _deg_part(dst, ones, z1).reshape(NC, NP, 1)
    y0 = _tc1(x, W0, degp)
    p = _agg128(src, dst, y0, z128)
    h, y1 = _tc2(p, y0, degp, b0.reshape(1, F), W1)
    q = _agg128(src, dst, y1, z128)
    z = _tc3(q, y1, h, degp, b1.reshape(1, F),
             jk2_Wih_f.T, jk2_Whh_f.T, (jk2_bih_f + jk2_bhh_f).reshape(1, 4 * F),
             jk2_Wih_b.T, jk2_Whh_b.T, (jk2_bih_b + jk2_bhh_b).reshape(1, 4 * F),
             jk2_att_W[:F].reshape(1, F), jk2_att_W[F:].reshape(1, F))
    r = _agg128(src, dst, z, z128)
    return _tc4(r, z, degp, W2, b2.reshape(1, NCLS))


# speedup vs baseline: 24.9001x; 2.2655x over previous
import functools

import jax
import jax.numpy as jnp
from jax import lax
from jax.experimental import pallas as pl
from jax.experimental.pallas import tpu as pltpu
from jax.experimental.pallas import tpu_sc as plsc

N = 10000
E = 320000
F = 128
NCLS = 40
NC, NS = 2, 16
NW = NC * NS
EPW = E // NW
CH = 80
ITERS = EPW // CH
NP = 10240
RPT = NP // NS
RB = 1000

@functools.lru_cache(maxsize=None)
def _mesh():
    return plsc.VectorSubcoreMesh(core_axis_name="c", subcore_axis_name="s")


@functools.lru_cache(maxsize=None)
def _make_agg(D):
    @functools.partial(
        pl.kernel,
        out_type=jax.ShapeDtypeStruct((NC, NP, D), jnp.float32),
        mesh=_mesh(),
        scratch_types=[
            pltpu.VMEM((EPW,), jnp.int32),
            pltpu.VMEM((EPW,), jnp.int32),
            pltpu.VMEM((CH,), jnp.int32),
            pltpu.VMEM((CH,), jnp.int32),
            pltpu.VMEM((CH, D), jnp.float32),
            pltpu.VMEM((CH, D), jnp.float32),
            pltpu.VMEM_SHARED((NP, D), jnp.float32),
            pltpu.SemaphoreType.DMA,
            pltpu.SemaphoreType.DMA,
        ],
    )
    def agg(src_hbm, dst_hbm, y_hbm, zero_hbm, part_hbm,
            src1d, dst1d, idx_da, idx_db, rows_a, rows_b, acc, sem_a, sem_b):
        cid = lax.axis_index("c")
        sid = lax.axis_index("s")
        wid = sid * NC + cid
        r0 = sid * RPT
        pltpu.sync_copy(zero_hbm.at[pl.ds(r0, RPT)], acc.at[pl.ds(r0, RPT)])
        pltpu.sync_copy(src_hbm.at[pl.ds(wid * EPW, EPW)], src1d)
        pltpu.sync_copy(dst_hbm.at[pl.ds(wid * EPW, EPW)], dst1d)
        plsc.subcore_barrier()

        def fill_idx(buf, g):
            for k in range(CH // 16):
                buf[pl.ds(k * 16, 16)] = dst1d[pl.ds(g * CH + k * 16, 16)]

        def gather(g, buf, sem):
            pltpu.async_copy(y_hbm.at[src1d.at[pl.ds(g * CH, CH)]], buf, sem)

        def wait_gather(buf, sem):
            pltpu.make_async_copy(y_hbm.at[pl.ds(0, CH)], buf, sem).wait()

        gather(0, rows_a, sem_a)
        fill_idx(idx_da, 0)

        @pl.loop(1, ITERS, step=2)
        def _pair(g):
            gather(g, rows_b, sem_b)
            fill_idx(idx_db, g)
            wait_gather(rows_a, sem_a)
            pltpu.sync_copy(rows_a, acc.at[idx_da], add=True)
            gather(g + 1, rows_a, sem_a)
            fill_idx(idx_da, g + 1)
            wait_gather(rows_b, sem_b)
            pltpu.sync_copy(rows_b, acc.at[idx_db], add=True)

        wait_gather(rows_a, sem_a)
        pltpu.sync_copy(rows_a, acc.at[idx_da], add=True)
        plsc.subcore_barrier()
        pltpu.sync_copy(acc.at[pl.ds(r0, RPT)], part_hbm.at[cid, pl.ds(r0, RPT)])

    return agg


def _agg128(src, dst, y, zero):
    return _make_agg(F)(src, dst, y, zero)


@functools.lru_cache(maxsize=None)
def _make_deg():
    @functools.partial(
        pl.kernel,
        out_type=jax.ShapeDtypeStruct((NC, NP), jnp.float32),
        mesh=_mesh(),
        scratch_types=[
            pltpu.VMEM((EPW,), jnp.int32),
            pltpu.VMEM((CH,), jnp.int32),
            pltpu.VMEM((CH,), jnp.float32),
            pltpu.VMEM_SHARED((NP,), jnp.float32),
        ],
    )
    def deg(dst_hbm, ones_hbm, zero_hbm, part_hbm, dst1d, idx_d, ones_v, acc):
        cid = lax.axis_index("c")
        sid = lax.axis_index("s")
        wid = sid * NC + cid
        r0 = sid * RPT
        pltpu.sync_copy(zero_hbm.at[pl.ds(r0, RPT)], acc.at[pl.ds(r0, RPT)])
        pltpu.sync_copy(dst_hbm.at[pl.ds(wid * EPW, EPW)], dst1d)
        pltpu.sync_copy(ones_hbm, ones_v)
        plsc.subcore_barrier()

        @pl.loop(0, ITERS)
        def _chunk(g):
            for k in range(CH // 16):
                idx_d[pl.ds(k * 16, 16)] = dst1d[pl.ds(g * CH + k * 16, 16)]
            pltpu.sync_copy(ones_v, acc.at[idx_d], add=True)

        plsc.subcore_barrier()
        pltpu.sync_copy(acc.at[pl.ds(r0, RPT)], part_hbm.at[cid, pl.ds(r0, RPT)])

    return deg


def _deg_part(dst, ones, zero):
    return _make_deg()(dst, ones, zero)


def _dinv_of(degp_ref):
    deg = degp_ref[0, :, 0:1] + degp_ref[1, :, 0:1] + 1.0
    return lax.rsqrt(deg)


def _k1(x_ref, w0_ref, degp_ref, y0_ref):
    dinv = _dinv_of(degp_ref)
    y0_ref[...] = jnp.dot(x_ref[...], w0_ref[...],
                          preferred_element_type=jnp.float32) * dinv


def _k2(p_ref, y0_ref, degp_ref, b0_ref, w1_ref, h_ref, y1_ref):
    dinv = _dinv_of(degp_ref)
    h = jnp.maximum(dinv * (p_ref[0] + p_ref[1] + y0_ref[...]) + b0_ref[...], 0.0)
    h_ref[...] = h
    y1_ref[...] = jnp.dot(h, w1_ref[...], preferred_element_type=jnp.float32) * dinv


def _lstm_step0(xt, WiT, b):
    g = jnp.dot(xt, WiT, preferred_element_type=jnp.float32) + b
    c = jax.nn.sigmoid(g[:, 0:F]) * jnp.tanh(g[:, 2 * F:3 * F])
    return jax.nn.sigmoid(g[:, 3 * F:4 * F]) * jnp.tanh(c), c


def _lstm_step(xt, hp, cp, WiT, WhT, b):
    g = (jnp.dot(xt, WiT, preferred_element_type=jnp.float32)
         + jnp.dot(hp, WhT, preferred_element_type=jnp.float32) + b)
    c = (jax.nn.sigmoid(g[:, F:2 * F]) * cp
         + jax.nn.sigmoid(g[:, 0:F]) * jnp.tanh(g[:, 2 * F:3 * F]))
    return jax.nn.sigmoid(g[:, 3 * F:4 * F]) * jnp.tanh(c), c


def _k3(q_ref, y1_ref, h_ref, degp_ref, b1_ref, wif_ref, whf_ref, bf_ref,
        wib_ref, whb_ref, bb_ref, awf_ref, awb_ref, z_ref):
    dinv = _dinv_of(degp_ref)
    h = h_ref[...]
    h2 = jnp.maximum(dinv * (q_ref[0] + q_ref[1] + y1_ref[...]) + b1_ref[...], 0.0)
    hf0, cf0 = _lstm_step0(h, wif_ref[...], bf_ref[...])
    hf1, _ = _lstm_step(h2, hf0, cf0, wif_ref[...], whf_ref[...], bf_ref[...])
    hb1, cb1 = _lstm_step0(h2, wib_ref[...], bb_ref[...])
    hb0, _ = _lstm_step(h, hb1, cb1, wib_ref[...], whb_ref[...], bb_ref[...])
    awf = awf_ref[...]
    awb = awb_ref[...]
    a0 = (jnp.sum(hf0 * awf, axis=1, keepdims=True)
          + jnp.sum(hb0 * awb, axis=1, keepdims=True))
    a1 = (jnp.sum(hf1 * awf, axis=1, keepdims=True)
          + jnp.sum(hb1 * awb, axis=1, keepdims=True))
    m = jnp.maximum(a0, a1)
    e0 = jnp.exp(a0 - m)
    e1 = jnp.exp(a1 - m)
    ho = (e0 * h + e1 * h2) / (e0 + e1)
    z_ref[...] = ho * dinv


def _k4(r_ref, z_ref, degp_ref, w2_ref, b2_ref, out_ref):
    dinv = _dinv_of(degp_ref)
    t = dinv * (r_ref[0] + r_ref[1] + z_ref[...])
    l = jnp.dot(t, w2_ref[...], preferred_element_type=jnp.float32) + b2_ref[...]
    m = jnp.max(l, axis=1, keepdims=True)
    lse = jnp.log(jnp.sum(jnp.exp(l - m), axis=1, keepdims=True))
    out_ref[...] = l - m - lse


def _row_spec(d):
    return pl.BlockSpec((RB, d), lambda r: (r, 0))


def _part_spec(d):
    return pl.BlockSpec((NC, RB, d), lambda r: (0, r, 0))


def _full_spec(a, b):
    return pl.BlockSpec((a, b), lambda r: (0, 0))


_GRID = (N // RB,)

_tc1 = pl.pallas_call(
    _k1, grid=_GRID,
    in_specs=[_row_spec(F), _full_spec(F, F), _part_spec(1)],
    out_specs=_row_spec(F),
    out_shape=jax.ShapeDtypeStruct((N, F), jnp.float32))

_tc2 = pl.pallas_call(
    _k2, grid=_GRID,
    in_specs=[_part_spec(F), _row_spec(F), _part_spec(1), _full_spec(1, F),
              _full_spec(F, F)],
    out_specs=[_row_spec(F), _row_spec(F)],
    out_shape=[jax.ShapeDtypeStruct((N, F), jnp.float32),
               jax.ShapeDtypeStruct((N, F), jnp.float32)])

_tc3 = pl.pallas_call(
    _k3, grid=_GRID,
    in_specs=[_part_spec(F), _row_spec(F), _row_spec(F), _part_spec(1),
              _full_spec(1, F),
              _full_spec(F, 4 * F), _full_spec(F, 4 * F), _full_spec(1, 4 * F),
              _full_spec(F, 4 * F), _full_spec(F, 4 * F), _full_spec(1, 4 * F),
              _full_spec(1, F), _full_spec(1, F)],
    out_specs=_row_spec(F),
    out_shape=jax.ShapeDtypeStruct((N, F), jnp.float32))

_tc4 = pl.pallas_call(
    _k4, grid=_GRID,
    in_specs=[_part_spec(F), _row_spec(F), _part_spec(1),
              _full_spec(F, NCLS), _full_spec(1, NCLS)],
    out_specs=pl.BlockSpec((RB, NCLS), lambda r: (r, 0)),
    out_shape=jax.ShapeDtypeStruct((N, NCLS), jnp.float32))


def kernel(x, edge_index, W0, b0, jk1_Wih_f, jk1_Whh_f, jk1_bih_f, jk1_bhh_f,
           jk1_Wih_b, jk1_Whh_b, jk1_bih_b, jk1_bhh_b, jk1_att_W, jk1_att_b,
           W1, b1, jk2_Wih_f, jk2_Whh_f, jk2_bih_f, jk2_bhh_f, jk2_Wih_b,
           jk2_Whh_b, jk2_bih_b, jk2_bhh_b, jk2_att_W, jk2_att_b, W2, b2):
    src = edge_index[0]
    dst = edge_index[1]
    z1 = jnp.zeros((NP,), jnp.float32)
    z128 = jnp.zeros((NP, F), jnp.float32)
    ones = jnp.ones((CH,), jnp.float32)

    degp = _deg_part(dst, ones, z1).reshape(NC, NP, 1)
    y0 = _tc1(x, W0, degp)
    p = _agg128(src, dst, y0, z128)
    h, y1 = _tc2(p, y0, degp, b0.reshape(1, F), W1)
    q = _agg128(src, dst, y1, z128)
    z = _tc3(q, y1, h, degp, b1.reshape(1, F),
             jk2_Wih_f.T, jk2_Whh_f.T, (jk2_bih_f + jk2_bhh_f).reshape(1, 4 * F),
             jk2_Wih_b.T, jk2_Whh_b.T, (jk2_bih_b + jk2_bhh_b).reshape(1, 4 * F),
             jk2_att_W[:F].reshape(1, F), jk2_att_W[F:].reshape(1, F))
    r = _agg128(src, dst, z, z128)
    return _tc4(r, z, degp, W2, b2.reshape(1, NCLS))


# speedup vs baseline: 26.5255x; 1.0653x over previous
import functools

import jax
import jax.numpy as jnp
from jax import lax
from jax.experimental import pallas as pl
from jax.experimental.pallas import tpu as pltpu
from jax.experimental.pallas import tpu_sc as plsc

N = 10000
E = 320000
F = 128
NCLS = 40
NC, NS = 2, 16
NW = NC * NS
EPW = E // NW
CH = 112
NFULL = EPW // CH
TAIL = EPW - NFULL * CH
ITERS = 125
CH_DEG = 80
NP = 10240
RPT = NP // NS
RB = 1000

@functools.lru_cache(maxsize=None)
def _mesh():
    return plsc.VectorSubcoreMesh(core_axis_name="c", subcore_axis_name="s")


@functools.lru_cache(maxsize=None)
def _make_agg(D):
    @functools.partial(
        pl.kernel,
        out_type=jax.ShapeDtypeStruct((NC, NP, D), jnp.float32),
        mesh=_mesh(),
        scratch_types=[
            pltpu.VMEM((EPW,), jnp.int32),
            pltpu.VMEM((EPW,), jnp.int32),
            pltpu.VMEM((CH,), jnp.int32),
            pltpu.VMEM((CH,), jnp.int32),
            pltpu.VMEM((CH, D), jnp.float32),
            pltpu.VMEM((CH, D), jnp.float32),
            pltpu.VMEM_SHARED((NP, D), jnp.float32),
            pltpu.SemaphoreType.DMA,
            pltpu.SemaphoreType.DMA,
        ],
    )
    def agg(src_hbm, dst_hbm, y_hbm, zero_hbm, part_hbm,
            src1d, dst1d, idx_da, idx_db, rows_a, rows_b, acc, sem_a, sem_b):
        cid = lax.axis_index("c")
        sid = lax.axis_index("s")
        wid = sid * NC + cid
        r0 = sid * RPT
        pltpu.sync_copy(zero_hbm.at[pl.ds(r0, RPT)], acc.at[pl.ds(r0, RPT)])
        pltpu.sync_copy(src_hbm.at[pl.ds(wid * EPW, EPW)], src1d)
        pltpu.sync_copy(dst_hbm.at[pl.ds(wid * EPW, EPW)], dst1d)
        plsc.subcore_barrier()

        def fill_idx(buf, g, n):
            for k in range(n // 16):
                buf[pl.ds(k * 16, 16)] = dst1d[pl.ds(g * CH + k * 16, 16)]

        def gather(g, buf, sem, n):
            pltpu.async_copy(
                y_hbm.at[src1d.at[pl.ds(g * CH, n)]], buf.at[pl.ds(0, n)], sem)

        def wait_gather(buf, sem, n):
            pltpu.make_async_copy(
                y_hbm.at[pl.ds(0, n)], buf.at[pl.ds(0, n)], sem).wait()

        def scatter(buf, idx, n):
            pltpu.sync_copy(buf.at[pl.ds(0, n)], acc.at[idx.at[pl.ds(0, n)]], add=True)

        gather(0, rows_a, sem_a, CH)
        fill_idx(idx_da, 0, CH)

        @pl.loop(1, NFULL, step=2)
        def _pair(g):
            gather(g, rows_b, sem_b, CH)
            fill_idx(idx_db, g, CH)
            wait_gather(rows_a, sem_a, CH)
            scatter(rows_a, idx_da, CH)
            gather(g + 1, rows_a, sem_a, CH)
            fill_idx(idx_da, g + 1, CH)
            wait_gather(rows_b, sem_b, CH)
            scatter(rows_b, idx_db, CH)

        pltpu.async_copy(
            y_hbm.at[src1d.at[pl.ds(NFULL * CH, TAIL)]],
            rows_b.at[pl.ds(0, TAIL)], sem_b)
        for k in range(TAIL // 16):
            idx_db[pl.ds(k * 16, 16)] = dst1d[pl.ds(NFULL * CH + k * 16, 16)]
        wait_gather(rows_a, sem_a, CH)
        scatter(rows_a, idx_da, CH)
        wait_gather(rows_b, sem_b, TAIL)
        scatter(rows_b, idx_db, TAIL)
        plsc.subcore_barrier()
        pltpu.sync_copy(acc.at[pl.ds(r0, RPT)], part_hbm.at[cid, pl.ds(r0, RPT)])

    return agg


def _agg128(src, dst, y, zero):
    return _make_agg(F)(src, dst, y, zero)


@functools.lru_cache(maxsize=None)
def _make_deg():
    @functools.partial(
        pl.kernel,
        out_type=jax.ShapeDtypeStruct((NC, NP), jnp.float32),
        mesh=_mesh(),
        scratch_types=[
            pltpu.VMEM((EPW,), jnp.int32),
            pltpu.VMEM((CH_DEG,), jnp.int32),
            pltpu.VMEM((CH_DEG,), jnp.float32),
            pltpu.VMEM_SHARED((NP,), jnp.float32),
        ],
    )
    def deg(dst_hbm, ones_hbm, zero_hbm, part_hbm, dst1d, idx_d, ones_v, acc):
        cid = lax.axis_index("c")
        sid = lax.axis_index("s")
        wid = sid * NC + cid
        r0 = sid * RPT
        pltpu.sync_copy(zero_hbm.at[pl.ds(r0, RPT)], acc.at[pl.ds(r0, RPT)])
        pltpu.sync_copy(dst_hbm.at[pl.ds(wid * EPW, EPW)], dst1d)
        pltpu.sync_copy(ones_hbm, ones_v)
        plsc.subcore_barrier()

        @pl.loop(0, ITERS)
        def _chunk(g):
            for k in range(CH_DEG // 16):
                idx_d[pl.ds(k * 16, 16)] = dst1d[pl.ds(g * CH_DEG + k * 16, 16)]
            pltpu.sync_copy(ones_v, acc.at[idx_d], add=True)

        plsc.subcore_barrier()
        pltpu.sync_copy(acc.at[pl.ds(r0, RPT)], part_hbm.at[cid, pl.ds(r0, RPT)])

    return deg


def _deg_part(dst, ones, zero):
    return _make_deg()(dst, ones, zero)


def _dinv_of(degp_ref):
    deg = degp_ref[0, :, 0:1] + degp_ref[1, :, 0:1] + 1.0
    return lax.rsqrt(deg)


def _k1(x_ref, w0_ref, degp_ref, y0_ref):
    dinv = _dinv_of(degp_ref)
    y0_ref[...] = jnp.dot(x_ref[...], w0_ref[...],
                          preferred_element_type=jnp.float32) * dinv


def _k2(p_ref, y0_ref, degp_ref, b0_ref, w1_ref, h_ref, y1_ref):
    dinv = _dinv_of(degp_ref)
    h = jnp.maximum(dinv * (p_ref[0] + p_ref[1] + y0_ref[...]) + b0_ref[...], 0.0)
    h_ref[...] = h
    y1_ref[...] = jnp.dot(h, w1_ref[...], preferred_element_type=jnp.float32) * dinv


def _lstm_step0(xt, WiT, b):
    g = jnp.dot(xt, WiT, preferred_element_type=jnp.float32) + b
    c = jax.nn.sigmoid(g[:, 0:F]) * jnp.tanh(g[:, 2 * F:3 * F])
    return jax.nn.sigmoid(g[:, 3 * F:4 * F]) * jnp.tanh(c), c


def _lstm_step(xt, hp, cp, WiT, WhT, b):
    g = (jnp.dot(xt, WiT, preferred_element_type=jnp.float32)
         + jnp.dot(hp, WhT, preferred_element_type=jnp.float32) + b)
    c = (jax.nn.sigmoid(g[:, F:2 * F]) * cp
         + jax.nn.sigmoid(g[:, 0:F]) * jnp.tanh(g[:, 2 * F:3 * F]))
    return jax.nn.sigmoid(g[:, 3 * F:4 * F]) * jnp.tanh(c), c


def _k3(q_ref, y1_ref, h_ref, degp_ref, b1_ref, wif_ref, whf_ref, bf_ref,
        wib_ref, whb_ref, bb_ref, awf_ref, awb_ref, z_ref):
    dinv = _dinv_of(degp_ref)
    h = h_ref[...]
    h2 = jnp.maximum(dinv * (q_ref[0] + q_ref[1] + y1_ref[...]) + b1_ref[...], 0.0)
    hf0, cf0 = _lstm_step0(h, wif_ref[...], bf_ref[...])
    hf1, _ = _lstm_step(h2, hf0, cf0, wif_ref[...], whf_ref[...], bf_ref[...])
    hb1, cb1 = _lstm_step0(h2, wib_ref[...], bb_ref[...])
    hb0, _ = _lstm_step(h, hb1, cb1, wib_ref[...], whb_ref[...], bb_ref[...])
    awf = awf_ref[...]
    awb = awb_ref[...]
    a0 = (jnp.sum(hf0 * awf, axis=1, keepdims=True)
          + jnp.sum(hb0 * awb, axis=1, keepdims=True))
    a1 = (jnp.sum(hf1 * awf, axis=1, keepdims=True)
          + jnp.sum(hb1 * awb, axis=1, keepdims=True))
    m = jnp.maximum(a0, a1)
    e0 = jnp.exp(a0 - m)
    e1 = jnp.exp(a1 - m)
    ho = (e0 * h + e1 * h2) / (e0 + e1)
    z_ref[...] = ho * dinv


def _k4(r_ref, z_ref, degp_ref, w2_ref, b2_ref, out_ref):
    dinv = _dinv_of(degp_ref)
    t = dinv * (r_ref[0] + r_ref[1] + z_ref[...])
    l = jnp.dot(t, w2_ref[...], preferred_element_type=jnp.float32) + b2_ref[...]
    m = jnp.max(l, axis=1, keepdims=True)
    lse = jnp.log(jnp.sum(jnp.exp(l - m), axis=1, keepdims=True))
    out_ref[...] = l - m - lse


def _row_spec(d):
    return pl.BlockSpec((RB, d), lambda r: (r, 0))


def _part_spec(d):
    return pl.BlockSpec((NC, RB, d), lambda r: (0, r, 0))


def _full_spec(a, b):
    return pl.BlockSpec((a, b), lambda r: (0, 0))


_GRID = (N // RB,)

_tc1 = pl.pallas_call(
    _k1, grid=_GRID,
    in_specs=[_row_spec(F), _full_spec(F, F), _part_spec(1)],
    out_specs=_row_spec(F),
    out_shape=jax.ShapeDtypeStruct((N, F), jnp.float32))

_tc2 = pl.pallas_call(
    _k2, grid=_GRID,
    in_specs=[_part_spec(F), _row_spec(F), _part_spec(1), _full_spec(1, F),
              _full_spec(F, F)],
    out_specs=[_row_spec(F), _row_spec(F)],
    out_shape=[jax.ShapeDtypeStruct((N, F), jnp.float32),
               jax.ShapeDtypeStruct((N, F), jnp.float32)])

_tc3 = pl.pallas_call(
    _k3, grid=_GRID,
    in_specs=[_part_spec(F), _row_spec(F), _row_spec(F), _part_spec(1),
              _full_spec(1, F),
              _full_spec(F, 4 * F), _full_spec(F, 4 * F), _full_spec(1, 4 * F),
              _full_spec(F, 4 * F), _full_spec(F, 4 * F), _full_spec(1, 4 * F),
              _full_spec(1, F), _full_spec(1, F)],
    out_specs=_row_spec(F),
    out_shape=jax.ShapeDtypeStruct((N, F), jnp.float32))

_tc4 = pl.pallas_call(
    _k4, grid=_GRID,
    in_specs=[_part_spec(F), _row_spec(F), _part_spec(1),
              _full_spec(F, NCLS), _full_spec(1, NCLS)],
    out_specs=pl.BlockSpec((RB, NCLS), lambda r: (r, 0)),
    out_shape=jax.ShapeDtypeStruct((N, NCLS), jnp.float32))


def kernel(x, edge_index, W0, b0, jk1_Wih_f, jk1_Whh_f, jk1_bih_f, jk1_bhh_f,
           jk1_Wih_b, jk1_Whh_b, jk1_bih_b, jk1_bhh_b, jk1_att_W, jk1_att_b,
           W1, b1, jk2_Wih_f, jk2_Whh_f, jk2_bih_f, jk2_bhh_f, jk2_Wih_b,
           jk2_Whh_b, jk2_bih_b, jk2_bhh_b, jk2_att_W, jk2_att_b, W2, b2):
    src = edge_index[0]
    dst = edge_index[1]
    z1 = jnp.zeros((NP,), jnp.float32)
    z128 = jnp.zeros((NP, F), jnp.float32)
    ones = jnp.ones((CH_DEG,), jnp.float32)

    degp = _deg_part(dst, ones, z1).reshape(NC, NP, 1)
    y0 = _tc1(x, W0, degp)
    p = _agg128(src, dst, y0, z128)
    h, y1 = _tc2(p, y0, degp, b0.reshape(1, F), W1)
    q = _agg128(src, dst, y1, z128)
    z = _tc3(q, y1, h, degp, b1.reshape(1, F),
             jk2_Wih_f.T, jk2_Whh_f.T, (jk2_bih_f + jk2_bhh_f).reshape(1, 4 * F),
             jk2_Wih_b.T, jk2_Whh_b.T, (jk2_bih_b + jk2_bhh_b).reshape(1, 4 * F),
             jk2_att_W[:F].reshape(1, F), jk2_att_W[F:].reshape(1, F))
    r = _agg128(src, dst, z, z128)
    return _tc4(r, z, degp, W2, b2.reshape(1, NCLS))
